# baseline (speedup 1.0000x reference)
import functools

import jax
import jax.numpy as jnp
from jax import lax
from jax.experimental import pallas as pl
from jax.experimental.pallas import tpu as pltpu
from jax.experimental.pallas import tpu_sc as plsc

_B, _N, _K = 4, 4096, 9
_EPS = 1e-5
_TN1 = 256
_TN = 128
_NC, _NS = 2, 16
_NW = _NC * _NS
_CH = 128
_F32 = jnp.float32
_BF16 = jnp.bfloat16


def _leaky(x):
    return jnp.where(x >= 0, x, 0.1 * x)


def _knn_body(xt_ref, xa_ref, sqr_ref, sqc_ref, out_ref):
    b = pl.program_id(0)
    xt = xt_ref[0]
    xa = xa_ref[0]
    sqr = sqr_ref[0]
    sqc = sqc_ref[0]
    prod = lax.dot_general(
        xt.astype(_BF16), xa.astype(_BF16),
        (((0,), (0,)), ((), ())), preferred_element_type=_F32)
    dist = (sqr + sqc) - 2.0 * prod
    iota = lax.broadcasted_iota(jnp.int32, (_TN1, _N), 1)
    iota16 = lax.broadcasted_iota(jnp.int32, (_TN1, 16), 1)

    def step(k, carry):
        neg, acc = carry
        m = jnp.max(neg, axis=1, keepdims=True)
        j = jnp.min(jnp.where(neg == m, iota, _N), axis=1, keepdims=True)
        neg = jnp.where(iota == j, -jnp.inf, neg)
        acc = acc + j * (iota16 == k).astype(jnp.int32)
        return neg, acc

    _, acc = lax.fori_loop(
        0, _K, step, (-dist, jnp.zeros((_TN1, 16), jnp.int32)))
    out_ref[0] = acc[:, :_K] + b * _N


def _knn(xyz, sq):
    return pl.pallas_call(
        _knn_body,
        grid=(_B, _N // _TN1),
        in_specs=[
            pl.BlockSpec((1, 3, _TN1), lambda b, i: (b, 0, i)),
            pl.BlockSpec((1, 3, _N), lambda b, i: (b, 0, 0)),
            pl.BlockSpec((1, _TN1, 1), lambda b, i: (b, i, 0)),
            pl.BlockSpec((1, 1, _N), lambda b, i: (b, 0, 0)),
        ],
        out_specs=pl.BlockSpec((1, _TN1, _K), lambda b, i: (b, i, 0)),
        out_shape=jax.ShapeDtypeStruct((_B, _N, _K), jnp.int32),
    )(xyz, xyz, sq[:, :, None], sq[:, None, :])


def _sc_gather(idx_flat, tables):
    rows_total = idx_flat.shape[0]
    n_tab = len(tables)
    per_w = rows_total // _NW
    steps = per_w // _CH
    mesh = plsc.VectorSubcoreMesh(core_axis_name="c", subcore_axis_name="s")
    out_type = tuple(
        jax.ShapeDtypeStruct((rows_total, t.shape[1]), t.dtype) for t in tables)
    scratch = ([pltpu.VMEM((_CH,), jnp.int32)]
               + [pltpu.VMEM((_CH, t.shape[1]), t.dtype) for t in tables]
               + [pltpu.SemaphoreType.DMA] * n_tab)

    def body(*refs):
        idx_hbm = refs[0]
        tabs = refs[1:1 + n_tab]
        outs = refs[1 + n_tab:1 + 2 * n_tab]
        idx_v = refs[1 + 2 * n_tab]
        bufs = refs[2 + 2 * n_tab:2 + 3 * n_tab]
        sems = refs[2 + 3 * n_tab:]
        wid = lax.axis_index("s") * _NC + lax.axis_index("c")
        base = wid * per_w

        def step(i, carry):
            off = base + i * _CH
            pltpu.sync_copy(idx_hbm.at[pl.ds(off, _CH)], idx_v)
            copies = [pltpu.async_copy(tabs[t].at[idx_v], bufs[t], sems[t])
                      for t in range(n_tab)]
            for c in copies:
                c.wait()
            for t in range(n_tab):
                pltpu.sync_copy(bufs[t], outs[t].at[pl.ds(off, _CH)])
            return carry

        lax.fori_loop(0, steps, step, 0)

    fn = pl.kernel(body, out_type=out_type, mesh=mesh, scratch_types=scratch,
                   compiler_params=pltpu.CompilerParams(
                       use_tc_tiling_on_sc=False))
    return fn(idx_flat, *tables)


def _wn_weights(w0, b0, w1, b1, w2, b2):
    w0p = jnp.pad(w0.T, ((0, 13), (0, 0))).astype(_BF16)
    return (w0p, b0.reshape(1, 8),
            w1.T.astype(_BF16), b1.reshape(1, 8),
            w2.T.astype(_BF16), b2.reshape(1, 16))


def _lin_weights(lin_w, lin_b, c_in, d_feat):
    w3 = lin_w.reshape(128, c_in, 16)
    lf = jnp.transpose(w3[:, 3:, :], (2, 1, 0))
    lf = jnp.pad(lf, ((0, 0), (0, d_feat - (c_in - 3)), (0, 0))).astype(_BF16)
    lx = jnp.transpose(w3[:, :3, :], (2, 1, 0))
    lx = jnp.pad(lx, ((0, 0), (0, 13), (0, 0))).astype(_BF16)
    return lf, lx, lin_b.reshape(1, 128)


def _agg(fg_ref, gg_ref, ctr, lf_ref, lx_ref, w0, b0, w1, b1, w2, b2, lb, s):
    d_feat = fg_ref.shape[3]

    def step(k, carry):
        uf, ux = carry
        fk = fg_ref[0, :, k, :]
        gkb = (gg_ref[0, :, k, :] - ctr).astype(_BF16)
        h = jnp.maximum(
            s * (jnp.dot(gkb, w0, preferred_element_type=_F32) + b0), 0.0)
        h = jnp.maximum(
            s * (jnp.dot(h.astype(_BF16), w1, preferred_element_type=_F32)
                 + b1), 0.0)
        wt = jnp.maximum(
            s * (jnp.dot(h.astype(_BF16), w2, preferred_element_type=_F32)
                 + b2), 0.0)
        wt32 = wt.astype(_BF16).astype(_F32)
        gk32 = gkb.astype(_F32)
        uf = tuple(uf[w] + fk * wt32[:, w:w + 1] for w in range(16))
        ux = tuple(ux[w] + gk32 * wt32[:, w:w + 1] for w in range(16))
        return uf, ux

    uf, ux = lax.fori_loop(
        0, _K, step,
        (tuple(jnp.zeros((_TN, d_feat), _F32) for _ in range(16)),
         tuple(jnp.zeros((_TN, 16), _F32) for _ in range(16))))
    acc = None
    for w in range(16):
        d = (jnp.dot(uf[w].astype(_BF16), lf_ref[w],
                     preferred_element_type=_F32)
             + jnp.dot(ux[w].astype(_BF16), lx_ref[w],
                       preferred_element_type=_F32))
        acc = d if acc is None else acc + d
    return _leaky(s * (acc + lb))


def _pc1_body(fg_ref, gg_ref, ctr_ref, lf_ref, lx_ref,
              w0_ref, b0_ref, w1_ref, b1_ref, w2_ref, b2_ref, lb_ref, out_ref):
    s = 1.0 / jnp.sqrt(jnp.float32(1.0 + _EPS))
    out = _agg(fg_ref, gg_ref, ctr_ref[0], lf_ref, lx_ref,
               w0_ref[...], b0_ref[...], w1_ref[...], b1_ref[...],
               w2_ref[...], b2_ref[...], lb_ref[...], s)
    out_ref[0] = out.astype(_BF16).astype(_F32)


def _pc2_body(fg_ref, gg_ref, ctr_ref, lf_ref, lx_ref,
              w0_ref, b0_ref, w1_ref, b1_ref, w2_ref, b2_ref, lb_ref,
              m0_ref, m0b_ref, m1_ref, m1b_ref, fcw_ref, fcb_ref,
              x_ref, fl_ref):
    s = 1.0 / jnp.sqrt(jnp.float32(1.0 + _EPS))
    out = _agg(fg_ref, gg_ref, ctr_ref[0], lf_ref, lx_ref,
               w0_ref[...], b0_ref[...], w1_ref[...], b1_ref[...],
               w2_ref[...], b2_ref[...], lb_ref[...], s)
    x1 = _leaky(s * (jnp.dot(out.astype(_BF16), m0_ref[...],
                             preferred_element_type=_F32) + m0b_ref[...]))
    x2 = _leaky(s * (jnp.dot(x1.astype(_BF16), m1_ref[...],
                             preferred_element_type=_F32) + m1b_ref[...]))
    fl = (jnp.dot(x2.astype(_BF16), fcw_ref[...],
                  preferred_element_type=_F32) + fcb_ref[...])
    x_ref[0] = x2
    fl_ref[0] = jnp.clip(fl, -200.0, 200.0)


def _full(shape):
    zeros = (0,) * len(shape)
    return pl.BlockSpec(shape, lambda b, i, z=zeros: z)


def _pc_specs(d_feat):
    in_specs = [
        pl.BlockSpec((1, _TN, _K, d_feat), lambda b, i: (b, i, 0, 0)),
        pl.BlockSpec((1, _TN, _K, 16), lambda b, i: (b, i, 0, 0)),
        pl.BlockSpec((1, _TN, 16), lambda b, i: (b, i, 0)),
        _full((16, d_feat, 128)),
        _full((16, 16, 128)),
        _full((16, 8)), _full((1, 8)),
        _full((8, 8)), _full((1, 8)),
        _full((8, 16)), _full((1, 16)),
        _full((1, 128)),
    ]
    return in_specs


def kernel(xyz, feats, cost_volume, flow, pc1_wn_w0, pc1_wn_b0, pc1_wn_w1,
           pc1_wn_b1, pc1_wn_w2, pc1_wn_b2, pc1_lin_w, pc1_lin_b, pc2_wn_w0,
           pc2_wn_b0, pc2_wn_w1, pc2_wn_b1, pc2_wn_w2, pc2_wn_b2, pc2_lin_w,
           pc2_lin_b, mlp0_w, mlp0_b, mlp1_w, mlp1_b, fc_w, fc_b):
    sq = jnp.sum(xyz * xyz, axis=1)
    idx = _knn(xyz, sq)
    idx_flat = idx.reshape(_B * _N * _K)

    xyzp = jnp.pad(jnp.transpose(xyz, (0, 2, 1)).reshape(_B * _N, 3),
                   ((0, 0), (0, 13)))
    pts = jnp.transpose(jnp.concatenate([feats, cost_volume, flow], axis=1),
                        (0, 2, 1)).reshape(_B * _N, 195)
    pts = pts.astype(_BF16).astype(_F32)
    ptsp = jnp.pad(pts, ((0, 0), (0, 13)))

    xyzg, fg1 = _sc_gather(idx_flat, (xyzp, ptsp))

    wn1 = _wn_weights(pc1_wn_w0, pc1_wn_b0, pc1_wn_w1, pc1_wn_b1,
                      pc1_wn_w2, pc1_wn_b2)
    lf1, lx1, lb1 = _lin_weights(pc1_lin_w, pc1_lin_b, 198, 208)
    wn2 = _wn_weights(pc2_wn_w0, pc2_wn_b0, pc2_wn_w1, pc2_wn_b1,
                      pc2_wn_w2, pc2_wn_b2)
    lf2, lx2, lb2 = _lin_weights(pc2_lin_w, pc2_lin_b, 131, 128)

    ctr = xyzp.reshape(_B, _N, 16)
    grid = (_B, _N // _TN)
    out1 = pl.pallas_call(
        _pc1_body,
        grid=grid,
        in_specs=_pc_specs(208),
        out_specs=pl.BlockSpec((1, _TN, 128), lambda b, i: (b, i, 0)),
        out_shape=jax.ShapeDtypeStruct((_B, _N, 128), _F32),
    )(fg1.reshape(_B, _N, _K, 208), xyzg.reshape(_B, _N, _K, 16), ctr,
      lf1, lx1, *wn1, lb1)

    (fg2,) = _sc_gather(idx_flat, (out1.reshape(_B * _N, 128),))

    m0 = mlp0_w.T.astype(_BF16)
    m1 = mlp1_w.T.astype(_BF16)
    fcw = fc_w.T.astype(_BF16)
    xo, flo = pl.pallas_call(
        _pc2_body,
        grid=grid,
        in_specs=_pc_specs(128) + [
            _full((128, 128)), _full((1, 128)),
            _full((128, 64)), _full((1, 64)),
            _full((64, 3)), _full((1, 3)),
        ],
        out_specs=[
            pl.BlockSpec((1, _TN, 64), lambda b, i: (b, i, 0)),
            pl.BlockSpec((1, _TN, 3), lambda b, i: (b, i, 0)),
        ],
        out_shape=[
            jax.ShapeDtypeStruct((_B, _N, 64), _F32),
            jax.ShapeDtypeStruct((_B, _N, 3), _F32),
        ],
    )(fg2.reshape(_B, _N, _K, 128), xyzg.reshape(_B, _N, _K, 16), ctr,
      lf2, lx2, *wn2, lb2,
      m0, mlp0_b.reshape(1, 128), m1, mlp1_b.reshape(1, 64),
      fcw, fc_b.reshape(1, 3))

    return (jnp.transpose(xo, (0, 2, 1)), jnp.transpose(flo, (0, 2, 1)))


# speedup vs baseline: 3.1009x; 3.1009x over previous
import functools

import jax
import jax.numpy as jnp
from jax import lax
from jax.experimental import pallas as pl
from jax.experimental.pallas import tpu as pltpu
from jax.experimental.pallas import tpu_sc as plsc

_B, _N, _K = 4, 4096, 9
_EPS = 1e-5
_TN1 = 256
_TN = 256
_NC, _NS = 2, 16
_NW = _NC * _NS
_CH = 128
_F32 = jnp.float32
_BF16 = jnp.bfloat16


def _leaky(x):
    return jnp.where(x >= 0, x, 0.1 * x)


def _knn_body(xt_ref, xa_ref, sqr_ref, sqc_ref, out_ref):
    b = pl.program_id(0)
    xt = xt_ref[0]
    xa = xa_ref[0]
    sqr = sqr_ref[0]
    sqc = sqc_ref[0]
    prod = lax.dot_general(
        xt.astype(_BF16), xa.astype(_BF16),
        (((0,), (0,)), ((), ())), preferred_element_type=_F32)
    dist = (sqr + sqc) - 2.0 * prod
    iota = lax.broadcasted_iota(jnp.int32, (_TN1, _N), 1)
    iota16 = lax.broadcasted_iota(jnp.int32, (_TN1, 16), 1)

    def step(k, carry):
        neg, acc = carry
        m = jnp.max(neg, axis=1, keepdims=True)
        j = jnp.min(jnp.where(neg == m, iota, _N), axis=1, keepdims=True)
        neg = jnp.where(iota == j, -jnp.inf, neg)
        acc = acc + j * (iota16 == k).astype(jnp.int32)
        return neg, acc

    _, acc = lax.fori_loop(
        0, _K, step, (-dist, jnp.zeros((_TN1, 16), jnp.int32)))
    out_ref[0] = acc[:, :_K] + b * _N


def _knn(xyz, sq):
    return pl.pallas_call(
        _knn_body,
        grid=(_B, _N // _TN1),
        in_specs=[
            pl.BlockSpec((1, 3, _TN1), lambda b, i: (b, 0, i)),
            pl.BlockSpec((1, 3, _N), lambda b, i: (b, 0, 0)),
            pl.BlockSpec((1, _TN1, 1), lambda b, i: (b, i, 0)),
            pl.BlockSpec((1, 1, _N), lambda b, i: (b, 0, 0)),
        ],
        out_specs=pl.BlockSpec((1, _TN1, _K), lambda b, i: (b, i, 0)),
        out_shape=jax.ShapeDtypeStruct((_B, _N, _K), jnp.int32),
    )(xyz, xyz, sq[:, :, None], sq[:, None, :])


def _sc_gather(idx_flat, tables):
    rows_total = idx_flat.shape[0]
    n_tab = len(tables)
    per_w = rows_total // _NW
    steps = per_w // _CH
    mesh = plsc.VectorSubcoreMesh(core_axis_name="c", subcore_axis_name="s")
    out_type = tuple(
        jax.ShapeDtypeStruct((rows_total, t.shape[1]), t.dtype) for t in tables)
    scratch = ([pltpu.VMEM((_CH,), jnp.int32)]
               + [pltpu.VMEM((_CH, t.shape[1]), t.dtype) for t in tables]
               + [pltpu.SemaphoreType.DMA] * n_tab)

    def body(*refs):
        idx_hbm = refs[0]
        tabs = refs[1:1 + n_tab]
        outs = refs[1 + n_tab:1 + 2 * n_tab]
        idx_v = refs[1 + 2 * n_tab]
        bufs = refs[2 + 2 * n_tab:2 + 3 * n_tab]
        sems = refs[2 + 3 * n_tab:]
        wid = lax.axis_index("s") * _NC + lax.axis_index("c")
        base = wid * per_w

        def step(i, carry):
            off = base + i * _CH
            pltpu.sync_copy(idx_hbm.at[pl.ds(off, _CH)], idx_v)
            copies = [pltpu.async_copy(tabs[t].at[idx_v], bufs[t], sems[t])
                      for t in range(n_tab)]
            for c in copies:
                c.wait()
            for t in range(n_tab):
                pltpu.sync_copy(bufs[t], outs[t].at[pl.ds(off, _CH)])
            return carry

        lax.fori_loop(0, steps, step, 0)

    fn = pl.kernel(body, out_type=out_type, mesh=mesh, scratch_types=scratch,
                   compiler_params=pltpu.CompilerParams(
                       use_tc_tiling_on_sc=False))
    return fn(idx_flat, *tables)


def _wn_weights(w0, b0, w1, b1, w2, b2):
    w0p = jnp.pad(w0.T, ((0, 13), (0, 0))).astype(_BF16)
    return (w0p, b0.reshape(1, 8),
            w1.T.astype(_BF16), b1.reshape(1, 8),
            w2.T.astype(_BF16), b2.reshape(1, 16))


def _lin_weights(lin_w, lin_b, c_in, d_feat):
    w3 = lin_w.reshape(128, c_in, 16)
    lf = jnp.transpose(w3[:, 3:, :], (2, 1, 0))
    lf = jnp.pad(lf, ((0, 0), (0, d_feat - (c_in - 3)), (0, 0))).astype(_BF16)
    lx = jnp.transpose(w3[:, :3, :], (2, 1, 0))
    lx = jnp.pad(lx, ((0, 0), (0, 13), (0, 0))).astype(_BF16)
    return lf, lx, lin_b.reshape(1, 128)


def _agg(fg_ref, gg_ref, ctr, lf_ref, lx_ref, w0, b0, w1, b1, w2, b2, lb, s):
    d_feat = fg_ref.shape[3]

    def step(k, carry):
        uf3, ux3 = carry
        fk = fg_ref[0, :, k, :]
        gkb = (gg_ref[0, :, k, :] - ctr).astype(_BF16)
        h = jnp.maximum(
            s * (jnp.dot(gkb, w0, preferred_element_type=_F32) + b0), 0.0)
        h = jnp.maximum(
            s * (jnp.dot(h.astype(_BF16), w1, preferred_element_type=_F32)
                 + b1), 0.0)
        wt = jnp.maximum(
            s * (jnp.dot(h.astype(_BF16), w2, preferred_element_type=_F32)
                 + b2), 0.0)
        wt32 = wt.astype(_BF16).astype(_F32)
        uf3 = uf3 + fk[:, None, :] * wt32[:, :, None]
        ux3 = ux3 + gkb.astype(_F32)[:, None, :] * wt32[:, :, None]
        return uf3, ux3

    uf3, ux3 = lax.fori_loop(
        0, _K, step,
        (jnp.zeros((_TN, 16, d_feat), _F32), jnp.zeros((_TN, 16, 16), _F32)))
    acc = None
    for w in range(16):
        d = (jnp.dot(uf3[:, w, :].astype(_BF16), lf_ref[w],
                     preferred_element_type=_F32)
             + jnp.dot(ux3[:, w, :].astype(_BF16), lx_ref[w],
                       preferred_element_type=_F32))
        acc = d if acc is None else acc + d
    return _leaky(s * (acc + lb))


def _pc1_body(fg_ref, gg_ref, ctr_ref, lf_ref, lx_ref,
              w0_ref, b0_ref, w1_ref, b1_ref, w2_ref, b2_ref, lb_ref, out_ref):
    s = 1.0 / jnp.sqrt(jnp.float32(1.0 + _EPS))
    out = _agg(fg_ref, gg_ref, ctr_ref[0], lf_ref, lx_ref,
               w0_ref[...], b0_ref[...], w1_ref[...], b1_ref[...],
               w2_ref[...], b2_ref[...], lb_ref[...], s)
    out_ref[0] = out.astype(_BF16).astype(_F32)


def _pc2_body(fg_ref, gg_ref, ctr_ref, lf_ref, lx_ref,
              w0_ref, b0_ref, w1_ref, b1_ref, w2_ref, b2_ref, lb_ref,
              m0_ref, m0b_ref, m1_ref, m1b_ref, fcw_ref, fcb_ref,
              x_ref, fl_ref):
    s = 1.0 / jnp.sqrt(jnp.float32(1.0 + _EPS))
    out = _agg(fg_ref, gg_ref, ctr_ref[0], lf_ref, lx_ref,
               w0_ref[...], b0_ref[...], w1_ref[...], b1_ref[...],
               w2_ref[...], b2_ref[...], lb_ref[...], s)
    x1 = _leaky(s * (jnp.dot(out.astype(_BF16), m0_ref[...],
                             preferred_element_type=_F32) + m0b_ref[...]))
    x2 = _leaky(s * (jnp.dot(x1.astype(_BF16), m1_ref[...],
                             preferred_element_type=_F32) + m1b_ref[...]))
    fl = (jnp.dot(x2.astype(_BF16), fcw_ref[...],
                  preferred_element_type=_F32) + fcb_ref[...])
    x_ref[0] = x2
    fl_ref[0] = jnp.clip(fl, -200.0, 200.0)


def _full(shape):
    zeros = (0,) * len(shape)
    return pl.BlockSpec(shape, lambda b, i, z=zeros: z)


def _pc_specs(d_feat):
    in_specs = [
        pl.BlockSpec((1, _TN, _K, d_feat), lambda b, i: (b, i, 0, 0)),
        pl.BlockSpec((1, _TN, _K, 16), lambda b, i: (b, i, 0, 0)),
        pl.BlockSpec((1, _TN, 16), lambda b, i: (b, i, 0)),
        _full((16, d_feat, 128)),
        _full((16, 16, 128)),
        _full((16, 8)), _full((1, 8)),
        _full((8, 8)), _full((1, 8)),
        _full((8, 16)), _full((1, 16)),
        _full((1, 128)),
    ]
    return in_specs


def kernel(xyz, feats, cost_volume, flow, pc1_wn_w0, pc1_wn_b0, pc1_wn_w1,
           pc1_wn_b1, pc1_wn_w2, pc1_wn_b2, pc1_lin_w, pc1_lin_b, pc2_wn_w0,
           pc2_wn_b0, pc2_wn_w1, pc2_wn_b1, pc2_wn_w2, pc2_wn_b2, pc2_lin_w,
           pc2_lin_b, mlp0_w, mlp0_b, mlp1_w, mlp1_b, fc_w, fc_b):
    sq = jnp.sum(xyz * xyz, axis=1)
    idx = _knn(xyz, sq)
    idx_flat = idx.reshape(_B * _N * _K)

    xyzp = jnp.pad(jnp.transpose(xyz, (0, 2, 1)).reshape(_B * _N, 3),
                   ((0, 0), (0, 13)))
    pts = jnp.transpose(jnp.concatenate([feats, cost_volume, flow], axis=1),
                        (0, 2, 1)).reshape(_B * _N, 195)
    pts = pts.astype(_BF16).astype(_F32)
    ptsp = jnp.pad(pts, ((0, 0), (0, 13)))

    xyzg, fg1 = _sc_gather(idx_flat, (xyzp, ptsp))

    wn1 = _wn_weights(pc1_wn_w0, pc1_wn_b0, pc1_wn_w1, pc1_wn_b1,
                      pc1_wn_w2, pc1_wn_b2)
    lf1, lx1, lb1 = _lin_weights(pc1_lin_w, pc1_lin_b, 198, 208)
    wn2 = _wn_weights(pc2_wn_w0, pc2_wn_b0, pc2_wn_w1, pc2_wn_b1,
                      pc2_wn_w2, pc2_wn_b2)
    lf2, lx2, lb2 = _lin_weights(pc2_lin_w, pc2_lin_b, 131, 128)

    ctr = xyzp.reshape(_B, _N, 16)
    grid = (_B, _N // _TN)
    out1 = pl.pallas_call(
        _pc1_body,
        grid=grid,
        in_specs=_pc_specs(208),
        out_specs=pl.BlockSpec((1, _TN, 128), lambda b, i: (b, i, 0)),
        out_shape=jax.ShapeDtypeStruct((_B, _N, 128), _F32),
    )(fg1.reshape(_B, _N, _K, 208), xyzg.reshape(_B, _N, _K, 16), ctr,
      lf1, lx1, *wn1, lb1)

    (fg2,) = _sc_gather(idx_flat, (out1.reshape(_B * _N, 128),))

    m0 = mlp0_w.T.astype(_BF16)
    m1 = mlp1_w.T.astype(_BF16)
    fcw = fc_w.T.astype(_BF16)
    xo, flo = pl.pallas_call(
        _pc2_body,
        grid=grid,
        in_specs=_pc_specs(128) + [
            _full((128, 128)), _full((1, 128)),
            _full((128, 64)), _full((1, 64)),
            _full((64, 3)), _full((1, 3)),
        ],
        out_specs=[
            pl.BlockSpec((1, _TN, 64), lambda b, i: (b, i, 0)),
            pl.BlockSpec((1, _TN, 3), lambda b, i: (b, i, 0)),
        ],
        out_shape=[
            jax.ShapeDtypeStruct((_B, _N, 64), _F32),
            jax.ShapeDtypeStruct((_B, _N, 3), _F32),
        ],
    )(fg2.reshape(_B, _N, _K, 128), xyzg.reshape(_B, _N, _K, 16), ctr,
      lf2, lx2, *wn2, lb2,
      m0, mlp0_b.reshape(1, 128), m1, mlp1_b.reshape(1, 64),
      fcw, fc_b.reshape(1, 3))

    return (jnp.transpose(xo, (0, 2, 1)), jnp.transpose(flo, (0, 2, 1)))


# speedup vs baseline: 5.3499x; 1.7253x over previous
import functools

import jax
import jax.numpy as jnp
from jax import lax
from jax.experimental import pallas as pl
from jax.experimental.pallas import tpu as pltpu
from jax.experimental.pallas import tpu_sc as plsc

_B, _N, _K = 4, 4096, 9
_EPS = 1e-5
_TN1 = 256
_TN = 128
_NC, _NS = 2, 16
_NW = _NC * _NS
_CH = 128
_F32 = jnp.float32
_BF16 = jnp.bfloat16


def _leaky(x):
    return jnp.where(x >= 0, x, 0.1 * x)


def _knn_body(xt_ref, xa_ref, sqr_ref, sqc_ref, out_ref):
    b = pl.program_id(0)
    xt = xt_ref[0]
    xa = xa_ref[0]
    sqr = sqr_ref[0]
    sqc = sqc_ref[0]
    prod = lax.dot_general(
        xt.astype(_BF16), xa.astype(_BF16),
        (((0,), (0,)), ((), ())), preferred_element_type=_F32)
    dist = (sqr + sqc) - 2.0 * prod
    iota = lax.broadcasted_iota(jnp.int32, (_TN1, _N), 1)
    iota16 = lax.broadcasted_iota(jnp.int32, (_TN1, 16), 1)

    def step(k, carry):
        neg, acc = carry
        m = jnp.max(neg, axis=1, keepdims=True)
        j = jnp.min(jnp.where(neg == m, iota, _N), axis=1, keepdims=True)
        neg = jnp.where(iota == j, -jnp.inf, neg)
        acc = acc + j * (iota16 == k).astype(jnp.int32)
        return neg, acc

    _, acc = lax.fori_loop(
        0, _K, step, (-dist, jnp.zeros((_TN1, 16), jnp.int32)))
    out_ref[0] = acc[:, :_K] + b * _N


def _knn(xyz, sq):
    return pl.pallas_call(
        _knn_body,
        grid=(_B, _N // _TN1),
        in_specs=[
            pl.BlockSpec((1, 3, _TN1), lambda b, i: (b, 0, i)),
            pl.BlockSpec((1, 3, _N), lambda b, i: (b, 0, 0)),
            pl.BlockSpec((1, _TN1, 1), lambda b, i: (b, i, 0)),
            pl.BlockSpec((1, 1, _N), lambda b, i: (b, 0, 0)),
        ],
        out_specs=pl.BlockSpec((1, _TN1, _K), lambda b, i: (b, i, 0)),
        out_shape=jax.ShapeDtypeStruct((_B, _N, _K), jnp.int32),
    )(xyz, xyz, sq[:, :, None], sq[:, None, :])


def _sc_gather(idx_flat, tables):
    rows_total = idx_flat.shape[0]
    n_tab = len(tables)
    per_w = rows_total // _NW
    steps = per_w // _CH
    mesh = plsc.VectorSubcoreMesh(core_axis_name="c", subcore_axis_name="s")
    out_type = tuple(
        jax.ShapeDtypeStruct((rows_total, t.shape[1]), t.dtype) for t in tables)
    scratch = ([pltpu.VMEM((_CH,), jnp.int32)]
               + [pltpu.VMEM((_CH, t.shape[1]), t.dtype) for t in tables]
               + [pltpu.SemaphoreType.DMA] * n_tab)

    def body(*refs):
        idx_hbm = refs[0]
        tabs = refs[1:1 + n_tab]
        outs = refs[1 + n_tab:1 + 2 * n_tab]
        idx_v = refs[1 + 2 * n_tab]
        bufs = refs[2 + 2 * n_tab:2 + 3 * n_tab]
        sems = refs[2 + 3 * n_tab:]
        wid = lax.axis_index("s") * _NC + lax.axis_index("c")
        base = wid * per_w

        def step(i, carry):
            off = base + i * _CH
            pltpu.sync_copy(idx_hbm.at[pl.ds(off, _CH)], idx_v)
            copies = [pltpu.async_copy(tabs[t].at[idx_v], bufs[t], sems[t])
                      for t in range(n_tab)]
            for c in copies:
                c.wait()
            for t in range(n_tab):
                pltpu.sync_copy(bufs[t], outs[t].at[pl.ds(off, _CH)])
            return carry

        lax.fori_loop(0, steps, step, 0)

    fn = pl.kernel(body, out_type=out_type, mesh=mesh, scratch_types=scratch,
                   compiler_params=pltpu.CompilerParams(
                       use_tc_tiling_on_sc=False))
    return fn(idx_flat, *tables)


def _wn_weights(w0, b0, w1, b1, w2, b2):
    w0p = jnp.pad(w0, ((0, 0), (0, 13))).astype(_BF16)
    return (w0p, b0.reshape(8, 1),
            w1.astype(_BF16), b1.reshape(8, 1),
            w2.astype(_BF16), b2.reshape(16, 1))


def _lin_weights(lin_w, lin_b, c_in, d_feat):
    w3 = lin_w.reshape(128, c_in, 16)
    lf = jnp.transpose(w3[:, 3:, :], (2, 0, 1))
    lf = jnp.pad(lf, ((0, 0), (0, 0), (0, d_feat - (c_in - 3)))).astype(_BF16)
    lx = jnp.transpose(w3[:, :3, :], (2, 0, 1))
    lx = jnp.pad(lx, ((0, 0), (0, 0), (0, 13))).astype(_BF16)
    return lf, lx, lin_b.reshape(128, 1)


def _agg(fg_ref, gg_ref, ctr_ref, lf_ref, lx_ref, w0, b0, w1, b1, w2, b2,
         lb, s):
    ctr_t = jnp.transpose(ctr_ref[0])
    uf = [None] * 16
    ux = [None] * 16
    for k in range(_K):
        fk_t = jnp.transpose(fg_ref[0, :, k, :])
        gkb = (jnp.transpose(gg_ref[0, :, k, :]) - ctr_t).astype(_BF16)
        h = jnp.maximum(
            s * (jnp.dot(w0, gkb, preferred_element_type=_F32) + b0), 0.0)
        h = jnp.maximum(
            s * (jnp.dot(w1, h.astype(_BF16), preferred_element_type=_F32)
                 + b1), 0.0)
        wt = jnp.maximum(
            s * (jnp.dot(w2, h.astype(_BF16), preferred_element_type=_F32)
                 + b2), 0.0)
        wt32 = wt.astype(_BF16).astype(_F32)
        gk32 = gkb.astype(_F32)
        for w in range(16):
            ws = wt32[w:w + 1, :]
            if k == 0:
                uf[w] = fk_t * ws
                ux[w] = gk32 * ws
            else:
                uf[w] = uf[w] + fk_t * ws
                ux[w] = ux[w] + gk32 * ws
    acc = None
    for w in range(16):
        d = (jnp.dot(lf_ref[w], uf[w].astype(_BF16),
                     preferred_element_type=_F32)
             + jnp.dot(lx_ref[w], ux[w].astype(_BF16),
                       preferred_element_type=_F32))
        acc = d if acc is None else acc + d
    return _leaky(s * (acc + lb))


def _pc1_body(fg_ref, gg_ref, ctr_ref, lf_ref, lx_ref,
              w0_ref, b0_ref, w1_ref, b1_ref, w2_ref, b2_ref, lb_ref, out_ref):
    s = 1.0 / jnp.sqrt(jnp.float32(1.0 + _EPS))
    out = _agg(fg_ref, gg_ref, ctr_ref, lf_ref, lx_ref,
               w0_ref[...], b0_ref[...], w1_ref[...], b1_ref[...],
               w2_ref[...], b2_ref[...], lb_ref[...], s)
    out_ref[0] = jnp.transpose(out).astype(_BF16).astype(_F32)


def _pc2_body(fg_ref, gg_ref, ctr_ref, lf_ref, lx_ref,
              w0_ref, b0_ref, w1_ref, b1_ref, w2_ref, b2_ref, lb_ref,
              m0_ref, m0b_ref, m1_ref, m1b_ref, fcw_ref, fcb_ref,
              x_ref, fl_ref):
    s = 1.0 / jnp.sqrt(jnp.float32(1.0 + _EPS))
    out = _agg(fg_ref, gg_ref, ctr_ref, lf_ref, lx_ref,
               w0_ref[...], b0_ref[...], w1_ref[...], b1_ref[...],
               w2_ref[...], b2_ref[...], lb_ref[...], s)
    x1 = _leaky(s * (jnp.dot(m0_ref[...], out.astype(_BF16),
                             preferred_element_type=_F32) + m0b_ref[...]))
    x2 = _leaky(s * (jnp.dot(m1_ref[...], x1.astype(_BF16),
                             preferred_element_type=_F32) + m1b_ref[...]))
    fl = (jnp.dot(fcw_ref[...], x2.astype(_BF16),
                  preferred_element_type=_F32) + fcb_ref[...])
    x_ref[0] = x2
    fl_ref[0] = jnp.clip(fl, -200.0, 200.0)


def _full(shape):
    zeros = (0,) * len(shape)
    return pl.BlockSpec(shape, lambda b, i, z=zeros: z)


def _pc_specs(d_feat):
    in_specs = [
        pl.BlockSpec((1, _TN, _K, d_feat), lambda b, i: (b, i, 0, 0)),
        pl.BlockSpec((1, _TN, _K, 16), lambda b, i: (b, i, 0, 0)),
        pl.BlockSpec((1, _TN, 16), lambda b, i: (b, i, 0)),
        _full((16, 128, d_feat)),
        _full((16, 128, 16)),
        _full((8, 16)), _full((8, 1)),
        _full((8, 8)), _full((8, 1)),
        _full((16, 8)), _full((16, 1)),
        _full((128, 1)),
    ]
    return in_specs


def kernel(xyz, feats, cost_volume, flow, pc1_wn_w0, pc1_wn_b0, pc1_wn_w1,
           pc1_wn_b1, pc1_wn_w2, pc1_wn_b2, pc1_lin_w, pc1_lin_b, pc2_wn_w0,
           pc2_wn_b0, pc2_wn_w1, pc2_wn_b1, pc2_wn_w2, pc2_wn_b2, pc2_lin_w,
           pc2_lin_b, mlp0_w, mlp0_b, mlp1_w, mlp1_b, fc_w, fc_b):
    sq = jnp.sum(xyz * xyz, axis=1)
    idx = _knn(xyz, sq)
    idx_flat = idx.reshape(_B * _N * _K)

    xyzp = jnp.pad(jnp.transpose(xyz, (0, 2, 1)).reshape(_B * _N, 3),
                   ((0, 0), (0, 13)))
    pts = jnp.transpose(jnp.concatenate([feats, cost_volume, flow], axis=1),
                        (0, 2, 1)).reshape(_B * _N, 195)
    pts = pts.astype(_BF16).astype(_F32)
    ptsp = jnp.pad(pts, ((0, 0), (0, 13)))

    xyzg, fg1 = _sc_gather(idx_flat, (xyzp, ptsp))

    wn1 = _wn_weights(pc1_wn_w0, pc1_wn_b0, pc1_wn_w1, pc1_wn_b1,
                      pc1_wn_w2, pc1_wn_b2)
    lf1, lx1, lb1 = _lin_weights(pc1_lin_w, pc1_lin_b, 198, 208)
    wn2 = _wn_weights(pc2_wn_w0, pc2_wn_b0, pc2_wn_w1, pc2_wn_b1,
                      pc2_wn_w2, pc2_wn_b2)
    lf2, lx2, lb2 = _lin_weights(pc2_lin_w, pc2_lin_b, 131, 128)

    ctr = xyzp.reshape(_B, _N, 16)
    grid = (_B, _N // _TN)
    out1 = pl.pallas_call(
        _pc1_body,
        grid=grid,
        in_specs=_pc_specs(208),
        out_specs=pl.BlockSpec((1, _TN, 128), lambda b, i: (b, i, 0)),
        out_shape=jax.ShapeDtypeStruct((_B, _N, 128), _F32),
    )(fg1.reshape(_B, _N, _K, 208), xyzg.reshape(_B, _N, _K, 16), ctr,
      lf1, lx1, *wn1, lb1)

    (fg2,) = _sc_gather(idx_flat, (out1.reshape(_B * _N, 128),))

    m0 = mlp0_w.astype(_BF16)
    m1 = mlp1_w.astype(_BF16)
    fcw = fc_w.astype(_BF16)
    xo, flo = pl.pallas_call(
        _pc2_body,
        grid=grid,
        in_specs=_pc_specs(128) + [
            _full((128, 128)), _full((128, 1)),
            _full((64, 128)), _full((64, 1)),
            _full((3, 64)), _full((3, 1)),
        ],
        out_specs=[
            pl.BlockSpec((1, 64, _TN), lambda b, i: (b, 0, i)),
            pl.BlockSpec((1, 3, _TN), lambda b, i: (b, 0, i)),
        ],
        out_shape=[
            jax.ShapeDtypeStruct((_B, 64, _N), _F32),
            jax.ShapeDtypeStruct((_B, 3, _N), _F32),
        ],
    )(fg2.reshape(_B, _N, _K, 128), xyzg.reshape(_B, _N, _K, 16), ctr,
      lf2, lx2, *wn2, lb2,
      m0, mlp0_b.reshape(128, 1), m1, mlp1_b.reshape(64, 1),
      fcw, fc_b.reshape(3, 1))

    return (xo, flo)


# speedup vs baseline: 5.5751x; 1.0421x over previous
import functools

import jax
import jax.numpy as jnp
from jax import lax
from jax.experimental import pallas as pl
from jax.experimental.pallas import tpu as pltpu
from jax.experimental.pallas import tpu_sc as plsc

_B, _N, _K = 4, 4096, 9
_EPS = 1e-5
_TN1 = 256
_TN = 256
_NC, _NS = 2, 16
_NW = _NC * _NS
_CH = 128
_F32 = jnp.float32
_BF16 = jnp.bfloat16


def _leaky(x):
    return jnp.where(x >= 0, x, 0.1 * x)


def _knn_body(xt_ref, xa_ref, sqr_ref, sqc_ref, out_ref):
    b = pl.program_id(0)
    xt = xt_ref[0]
    xa = xa_ref[0]
    sqr = sqr_ref[0]
    sqc = sqc_ref[0]
    prod = lax.dot_general(
        xt.astype(_BF16), xa.astype(_BF16),
        (((0,), (0,)), ((), ())), preferred_element_type=_F32)
    dist = (sqr + sqc) - 2.0 * prod
    iota = lax.broadcasted_iota(jnp.int32, (_TN1, _N), 1)
    iota16 = lax.broadcasted_iota(jnp.int32, (_TN1, 16), 1)

    def step(k, carry):
        neg, acc = carry
        m = jnp.max(neg, axis=1, keepdims=True)
        j = jnp.min(jnp.where(neg == m, iota, _N), axis=1, keepdims=True)
        neg = jnp.where(iota == j, -jnp.inf, neg)
        acc = acc + j * (iota16 == k).astype(jnp.int32)
        return neg, acc

    _, acc = lax.fori_loop(
        0, _K, step, (-dist, jnp.zeros((_TN1, 16), jnp.int32)))
    out_ref[0] = acc[:, :_K] + b * _N


def _knn(xyz, sq):
    return pl.pallas_call(
        _knn_body,
        grid=(_B, _N // _TN1),
        in_specs=[
            pl.BlockSpec((1, 3, _TN1), lambda b, i: (b, 0, i)),
            pl.BlockSpec((1, 3, _N), lambda b, i: (b, 0, 0)),
            pl.BlockSpec((1, _TN1, 1), lambda b, i: (b, i, 0)),
            pl.BlockSpec((1, 1, _N), lambda b, i: (b, 0, 0)),
        ],
        out_specs=pl.BlockSpec((1, _TN1, _K), lambda b, i: (b, i, 0)),
        out_shape=jax.ShapeDtypeStruct((_B, _N, _K), jnp.int32),
    )(xyz, xyz, sq[:, :, None], sq[:, None, :])


def _sc_gather(idx_flat, tables):
    rows_total = idx_flat.shape[0]
    n_tab = len(tables)
    per_w = rows_total // _NW
    steps = per_w // _CH
    mesh = plsc.VectorSubcoreMesh(core_axis_name="c", subcore_axis_name="s")
    out_type = tuple(
        jax.ShapeDtypeStruct((rows_total, t.shape[1]), t.dtype) for t in tables)
    scratch = ([pltpu.VMEM((_CH,), jnp.int32)]
               + [pltpu.VMEM((_CH, t.shape[1]), t.dtype) for t in tables]
               + [pltpu.SemaphoreType.DMA] * n_tab)

    def body(*refs):
        idx_hbm = refs[0]
        tabs = refs[1:1 + n_tab]
        outs = refs[1 + n_tab:1 + 2 * n_tab]
        idx_v = refs[1 + 2 * n_tab]
        bufs = refs[2 + 2 * n_tab:2 + 3 * n_tab]
        sems = refs[2 + 3 * n_tab:]
        wid = lax.axis_index("s") * _NC + lax.axis_index("c")
        base = wid * per_w

        def step(i, carry):
            off = base + i * _CH
            pltpu.sync_copy(idx_hbm.at[pl.ds(off, _CH)], idx_v)
            copies = [pltpu.async_copy(tabs[t].at[idx_v], bufs[t], sems[t])
                      for t in range(n_tab)]
            for c in copies:
                c.wait()
            for t in range(n_tab):
                pltpu.sync_copy(bufs[t], outs[t].at[pl.ds(off, _CH)])
            return carry

        lax.fori_loop(0, steps, step, 0)

    fn = pl.kernel(body, out_type=out_type, mesh=mesh, scratch_types=scratch,
                   compiler_params=pltpu.CompilerParams(
                       use_tc_tiling_on_sc=False))
    return fn(idx_flat, *tables)


def _wn_weights(w0, b0, w1, b1, w2, b2):
    w0p = jnp.pad(w0, ((0, 0), (0, 13))).astype(_BF16)
    return (w0p, b0.reshape(8, 1),
            w1.astype(_BF16), b1.reshape(8, 1),
            w2.astype(_BF16), b2.reshape(16, 1))


def _lin_weights(lin_w, lin_b, c_in, d_feat):
    w3 = lin_w.reshape(128, c_in, 16)
    lf = jnp.transpose(w3[:, 3:, :], (2, 0, 1))
    lf = jnp.pad(lf, ((0, 0), (0, 0), (0, d_feat - (c_in - 3)))).astype(_BF16)
    lx = jnp.transpose(w3[:, :3, :], (2, 0, 1))
    lx = jnp.pad(lx, ((0, 0), (0, 0), (0, 13))).astype(_BF16)
    return lf, lx, lin_b.reshape(128, 1)


def _agg(fg_ref, gg_ref, ctr_ref, lf_ref, lx_ref, w0, b0, w1, b1, w2, b2,
         lb, s):
    ctr_t = jnp.transpose(ctr_ref[0])
    uf = [None] * 16
    ux = [None] * 16
    for k in range(_K):
        fk_t = jnp.transpose(fg_ref[0, :, k, :])
        gkb = (jnp.transpose(gg_ref[0, :, k, :]) - ctr_t).astype(_BF16)
        h = jnp.maximum(
            s * (jnp.dot(w0, gkb, preferred_element_type=_F32) + b0), 0.0)
        h = jnp.maximum(
            s * (jnp.dot(w1, h.astype(_BF16), preferred_element_type=_F32)
                 + b1), 0.0)
        wt = jnp.maximum(
            s * (jnp.dot(w2, h.astype(_BF16), preferred_element_type=_F32)
                 + b2), 0.0)
        wt32 = wt.astype(_BF16).astype(_F32)
        gk32 = gkb.astype(_F32)
        for w in range(16):
            ws = wt32[w:w + 1, :]
            if k == 0:
                uf[w] = fk_t * ws
                ux[w] = gk32 * ws
            else:
                uf[w] = uf[w] + fk_t * ws
                ux[w] = ux[w] + gk32 * ws
    acc = None
    for w in range(16):
        d = (jnp.dot(lf_ref[w], uf[w].astype(_BF16),
                     preferred_element_type=_F32)
             + jnp.dot(lx_ref[w], ux[w].astype(_BF16),
                       preferred_element_type=_F32))
        acc = d if acc is None else acc + d
    return _leaky(s * (acc + lb))


def _pc1_body(fg_ref, gg_ref, ctr_ref, lf_ref, lx_ref,
              w0_ref, b0_ref, w1_ref, b1_ref, w2_ref, b2_ref, lb_ref, out_ref):
    s = 1.0 / jnp.sqrt(jnp.float32(1.0 + _EPS))
    out = _agg(fg_ref, gg_ref, ctr_ref, lf_ref, lx_ref,
               w0_ref[...], b0_ref[...], w1_ref[...], b1_ref[...],
               w2_ref[...], b2_ref[...], lb_ref[...], s)
    out_ref[0] = jnp.transpose(out).astype(_BF16).astype(_F32)


def _pc2_body(fg_ref, gg_ref, ctr_ref, lf_ref, lx_ref,
              w0_ref, b0_ref, w1_ref, b1_ref, w2_ref, b2_ref, lb_ref,
              m0_ref, m0b_ref, m1_ref, m1b_ref, fcw_ref, fcb_ref,
              x_ref, fl_ref):
    s = 1.0 / jnp.sqrt(jnp.float32(1.0 + _EPS))
    out = _agg(fg_ref, gg_ref, ctr_ref, lf_ref, lx_ref,
               w0_ref[...], b0_ref[...], w1_ref[...], b1_ref[...],
               w2_ref[...], b2_ref[...], lb_ref[...], s)
    x1 = _leaky(s * (jnp.dot(m0_ref[...], out.astype(_BF16),
                             preferred_element_type=_F32) + m0b_ref[...]))
    x2 = _leaky(s * (jnp.dot(m1_ref[...], x1.astype(_BF16),
                             preferred_element_type=_F32) + m1b_ref[...]))
    fl = (jnp.dot(fcw_ref[...], x2.astype(_BF16),
                  preferred_element_type=_F32) + fcb_ref[...])
    x_ref[0] = x2
    fl_ref[0] = jnp.clip(fl, -200.0, 200.0)


def _full(shape):
    zeros = (0,) * len(shape)
    return pl.BlockSpec(shape, lambda b, i, z=zeros: z)


def _pc_specs(d_feat):
    in_specs = [
        pl.BlockSpec((1, _TN, _K, d_feat), lambda b, i: (b, i, 0, 0)),
        pl.BlockSpec((1, _TN, _K, 16), lambda b, i: (b, i, 0, 0)),
        pl.BlockSpec((1, _TN, 16), lambda b, i: (b, i, 0)),
        _full((16, 128, d_feat)),
        _full((16, 128, 16)),
        _full((8, 16)), _full((8, 1)),
        _full((8, 8)), _full((8, 1)),
        _full((16, 8)), _full((16, 1)),
        _full((128, 1)),
    ]
    return in_specs


def kernel(xyz, feats, cost_volume, flow, pc1_wn_w0, pc1_wn_b0, pc1_wn_w1,
           pc1_wn_b1, pc1_wn_w2, pc1_wn_b2, pc1_lin_w, pc1_lin_b, pc2_wn_w0,
           pc2_wn_b0, pc2_wn_w1, pc2_wn_b1, pc2_wn_w2, pc2_wn_b2, pc2_lin_w,
           pc2_lin_b, mlp0_w, mlp0_b, mlp1_w, mlp1_b, fc_w, fc_b):
    sq = jnp.sum(xyz * xyz, axis=1)
    idx = _knn(xyz, sq)
    idx_flat = idx.reshape(_B * _N * _K)

    xyzp = jnp.pad(jnp.transpose(xyz, (0, 2, 1)).reshape(_B * _N, 3),
                   ((0, 0), (0, 13)))
    pts = jnp.transpose(jnp.concatenate([feats, cost_volume, flow], axis=1),
                        (0, 2, 1)).reshape(_B * _N, 195)
    pts = pts.astype(_BF16).astype(_F32)
    ptsp = jnp.pad(pts, ((0, 0), (0, 13)))

    xyzg, fg1 = _sc_gather(idx_flat, (xyzp, ptsp))

    wn1 = _wn_weights(pc1_wn_w0, pc1_wn_b0, pc1_wn_w1, pc1_wn_b1,
                      pc1_wn_w2, pc1_wn_b2)
    lf1, lx1, lb1 = _lin_weights(pc1_lin_w, pc1_lin_b, 198, 208)
    wn2 = _wn_weights(pc2_wn_w0, pc2_wn_b0, pc2_wn_w1, pc2_wn_b1,
                      pc2_wn_w2, pc2_wn_b2)
    lf2, lx2, lb2 = _lin_weights(pc2_lin_w, pc2_lin_b, 131, 128)

    ctr = xyzp.reshape(_B, _N, 16)
    grid = (_B, _N // _TN)
    out1 = pl.pallas_call(
        _pc1_body,
        grid=grid,
        in_specs=_pc_specs(208),
        out_specs=pl.BlockSpec((1, _TN, 128), lambda b, i: (b, i, 0)),
        out_shape=jax.ShapeDtypeStruct((_B, _N, 128), _F32),
    )(fg1.reshape(_B, _N, _K, 208), xyzg.reshape(_B, _N, _K, 16), ctr,
      lf1, lx1, *wn1, lb1)

    (fg2,) = _sc_gather(idx_flat, (out1.reshape(_B * _N, 128),))

    m0 = mlp0_w.astype(_BF16)
    m1 = mlp1_w.astype(_BF16)
    fcw = fc_w.astype(_BF16)
    xo, flo = pl.pallas_call(
        _pc2_body,
        grid=grid,
        in_specs=_pc_specs(128) + [
            _full((128, 128)), _full((128, 1)),
            _full((64, 128)), _full((64, 1)),
            _full((3, 64)), _full((3, 1)),
        ],
        out_specs=[
            pl.BlockSpec((1, 64, _TN), lambda b, i: (b, 0, i)),
            pl.BlockSpec((1, 3, _TN), lambda b, i: (b, 0, i)),
        ],
        out_shape=[
            jax.ShapeDtypeStruct((_B, 64, _N), _F32),
            jax.ShapeDtypeStruct((_B, 3, _N), _F32),
        ],
    )(fg2.reshape(_B, _N, _K, 128), xyzg.reshape(_B, _N, _K, 16), ctr,
      lf2, lx2, *wn2, lb2,
      m0, mlp0_b.reshape(128, 1), m1, mlp1_b.reshape(64, 1),
      fcw, fc_b.reshape(3, 1))

    return (xo, flo)


# speedup vs baseline: 5.6360x; 1.0109x over previous
import functools

import jax
import jax.numpy as jnp
from jax import lax
from jax.experimental import pallas as pl
from jax.experimental.pallas import tpu as pltpu
from jax.experimental.pallas import tpu_sc as plsc

_B, _N, _K = 4, 4096, 9
_EPS = 1e-5
_TN1 = 512
_TN = 256
_NC, _NS = 2, 16
_NW = _NC * _NS
_CH = 128
_F32 = jnp.float32
_BF16 = jnp.bfloat16


def _leaky(x):
    return jnp.where(x >= 0, x, 0.1 * x)


def _knn_body(xt_ref, xa_ref, sqr_ref, sqc_ref, out_ref):
    b = pl.program_id(0)
    xt = xt_ref[0]
    xa = xa_ref[0]
    sqr = sqr_ref[0]
    sqc = sqc_ref[0]
    prod = lax.dot_general(
        xt.astype(_BF16), xa.astype(_BF16),
        (((0,), (0,)), ((), ())), preferred_element_type=_F32)
    dist = (sqr + sqc) - 2.0 * prod
    iota = lax.broadcasted_iota(jnp.int32, (_TN1, _N), 1)
    iota16 = lax.broadcasted_iota(jnp.int32, (_TN1, 16), 1)

    def step(k, carry):
        neg, acc = carry
        m = jnp.max(neg, axis=1, keepdims=True)
        j = jnp.min(jnp.where(neg == m, iota, _N), axis=1, keepdims=True)
        neg = jnp.where(iota == j, -jnp.inf, neg)
        acc = acc + j * (iota16 == k).astype(jnp.int32)
        return neg, acc

    _, acc = lax.fori_loop(
        0, _K, step, (-dist, jnp.zeros((_TN1, 16), jnp.int32)))
    out_ref[0] = acc[:, :_K] + b * _N


def _knn(xyz, sq):
    return pl.pallas_call(
        _knn_body,
        grid=(_B, _N // _TN1),
        in_specs=[
            pl.BlockSpec((1, 3, _TN1), lambda b, i: (b, 0, i)),
            pl.BlockSpec((1, 3, _N), lambda b, i: (b, 0, 0)),
            pl.BlockSpec((1, _TN1, 1), lambda b, i: (b, i, 0)),
            pl.BlockSpec((1, 1, _N), lambda b, i: (b, 0, 0)),
        ],
        out_specs=pl.BlockSpec((1, _TN1, _K), lambda b, i: (b, i, 0)),
        out_shape=jax.ShapeDtypeStruct((_B, _N, _K), jnp.int32),
    )(xyz, xyz, sq[:, :, None], sq[:, None, :])


def _sc_gather(idx_flat, tables):
    rows_total = idx_flat.shape[0]
    n_tab = len(tables)
    per_w = rows_total // _NW
    steps = per_w // _CH
    mesh = plsc.VectorSubcoreMesh(core_axis_name="c", subcore_axis_name="s")
    out_type = tuple(
        jax.ShapeDtypeStruct((rows_total, t.shape[1]), t.dtype) for t in tables)
    scratch = ([pltpu.VMEM((_CH,), jnp.int32)]
               + [pltpu.VMEM((_CH, t.shape[1]), t.dtype) for t in tables]
               + [pltpu.SemaphoreType.DMA] * n_tab)

    def body(*refs):
        idx_hbm = refs[0]
        tabs = refs[1:1 + n_tab]
        outs = refs[1 + n_tab:1 + 2 * n_tab]
        idx_v = refs[1 + 2 * n_tab]
        bufs = refs[2 + 2 * n_tab:2 + 3 * n_tab]
        sems = refs[2 + 3 * n_tab:]
        wid = lax.axis_index("s") * _NC + lax.axis_index("c")
        base = wid * per_w

        def step(i, carry):
            off = base + i * _CH
            pltpu.sync_copy(idx_hbm.at[pl.ds(off, _CH)], idx_v)
            copies = [pltpu.async_copy(tabs[t].at[idx_v], bufs[t], sems[t])
                      for t in range(n_tab)]
            for c in copies:
                c.wait()
            for t in range(n_tab):
                pltpu.sync_copy(bufs[t], outs[t].at[pl.ds(off, _CH)])
            return carry

        lax.fori_loop(0, steps, step, 0)

    fn = pl.kernel(body, out_type=out_type, mesh=mesh, scratch_types=scratch,
                   compiler_params=pltpu.CompilerParams(
                       use_tc_tiling_on_sc=False))
    return fn(idx_flat, *tables)


def _wn_weights(w0, b0, w1, b1, w2, b2):
    w0p = jnp.pad(w0, ((0, 0), (0, 13))).astype(_BF16)
    return (w0p, b0.reshape(8, 1),
            w1.astype(_BF16), b1.reshape(8, 1),
            w2.astype(_BF16), b2.reshape(16, 1))


def _lin_weights(lin_w, lin_b, c_in, d_feat):
    w3 = lin_w.reshape(128, c_in, 16)
    lf = jnp.transpose(w3[:, 3:, :], (2, 0, 1))
    lf = jnp.pad(lf, ((0, 0), (0, 0), (0, d_feat - (c_in - 3)))).astype(_BF16)
    lx = jnp.transpose(w3[:, :3, :], (2, 0, 1))
    lx = jnp.pad(lx, ((0, 0), (0, 0), (0, 13))).astype(_BF16)
    return lf, lx, lin_b.reshape(128, 1)


def _agg(fg_ref, gg_ref, ctr_ref, lf_ref, lx_ref, w0, b0, w1, b1, w2, b2,
         lb, s):
    ctr_t = jnp.transpose(ctr_ref[0])
    uf = [None] * 16
    ux = [None] * 16
    for k in range(_K):
        fk_t = jnp.transpose(fg_ref[0, :, k, :])
        gkb = (jnp.transpose(gg_ref[0, :, k, :]) - ctr_t).astype(_BF16)
        h = jnp.maximum(
            s * (jnp.dot(w0, gkb, preferred_element_type=_F32) + b0), 0.0)
        h = jnp.maximum(
            s * (jnp.dot(w1, h.astype(_BF16), preferred_element_type=_F32)
                 + b1), 0.0)
        wt = jnp.maximum(
            s * (jnp.dot(w2, h.astype(_BF16), preferred_element_type=_F32)
                 + b2), 0.0)
        wt32 = wt.astype(_BF16).astype(_F32)
        gk32 = gkb.astype(_F32)
        for w in range(16):
            ws = wt32[w:w + 1, :]
            if k == 0:
                uf[w] = fk_t * ws
                ux[w] = gk32 * ws
            else:
                uf[w] = uf[w] + fk_t * ws
                ux[w] = ux[w] + gk32 * ws
    acc = None
    for w in range(16):
        d = (jnp.dot(lf_ref[w], uf[w].astype(_BF16),
                     preferred_element_type=_F32)
             + jnp.dot(lx_ref[w], ux[w].astype(_BF16),
                       preferred_element_type=_F32))
        acc = d if acc is None else acc + d
    return _leaky(s * (acc + lb))


def _pc1_body(fg_ref, gg_ref, ctr_ref, lf_ref, lx_ref,
              w0_ref, b0_ref, w1_ref, b1_ref, w2_ref, b2_ref, lb_ref, out_ref):
    s = 1.0 / jnp.sqrt(jnp.float32(1.0 + _EPS))
    out = _agg(fg_ref, gg_ref, ctr_ref, lf_ref, lx_ref,
               w0_ref[...], b0_ref[...], w1_ref[...], b1_ref[...],
               w2_ref[...], b2_ref[...], lb_ref[...], s)
    out_ref[0] = jnp.transpose(out).astype(_BF16).astype(_F32)


def _pc2_body(fg_ref, gg_ref, ctr_ref, lf_ref, lx_ref,
              w0_ref, b0_ref, w1_ref, b1_ref, w2_ref, b2_ref, lb_ref,
              m0_ref, m0b_ref, m1_ref, m1b_ref, fcw_ref, fcb_ref,
              x_ref, fl_ref):
    s = 1.0 / jnp.sqrt(jnp.float32(1.0 + _EPS))
    out = _agg(fg_ref, gg_ref, ctr_ref, lf_ref, lx_ref,
               w0_ref[...], b0_ref[...], w1_ref[...], b1_ref[...],
               w2_ref[...], b2_ref[...], lb_ref[...], s)
    x1 = _leaky(s * (jnp.dot(m0_ref[...], out.astype(_BF16),
                             preferred_element_type=_F32) + m0b_ref[...]))
    x2 = _leaky(s * (jnp.dot(m1_ref[...], x1.astype(_BF16),
                             preferred_element_type=_F32) + m1b_ref[...]))
    fl = (jnp.dot(fcw_ref[...], x2.astype(_BF16),
                  preferred_element_type=_F32) + fcb_ref[...])
    x_ref[0] = x2
    fl_ref[0] = jnp.clip(fl, -200.0, 200.0)


def _full(shape):
    zeros = (0,) * len(shape)
    return pl.BlockSpec(shape, lambda b, i, z=zeros: z)


def _pc_specs(d_feat):
    in_specs = [
        pl.BlockSpec((1, _TN, _K, d_feat), lambda b, i: (b, i, 0, 0)),
        pl.BlockSpec((1, _TN, _K, 16), lambda b, i: (b, i, 0, 0)),
        pl.BlockSpec((1, _TN, 16), lambda b, i: (b, i, 0)),
        _full((16, 128, d_feat)),
        _full((16, 128, 16)),
        _full((8, 16)), _full((8, 1)),
        _full((8, 8)), _full((8, 1)),
        _full((16, 8)), _full((16, 1)),
        _full((128, 1)),
    ]
    return in_specs


def kernel(xyz, feats, cost_volume, flow, pc1_wn_w0, pc1_wn_b0, pc1_wn_w1,
           pc1_wn_b1, pc1_wn_w2, pc1_wn_b2, pc1_lin_w, pc1_lin_b, pc2_wn_w0,
           pc2_wn_b0, pc2_wn_w1, pc2_wn_b1, pc2_wn_w2, pc2_wn_b2, pc2_lin_w,
           pc2_lin_b, mlp0_w, mlp0_b, mlp1_w, mlp1_b, fc_w, fc_b):
    sq = jnp.sum(xyz * xyz, axis=1)
    idx = _knn(xyz, sq)
    idx_flat = idx.reshape(_B * _N * _K)

    xyzp = jnp.pad(jnp.transpose(xyz, (0, 2, 1)).reshape(_B * _N, 3),
                   ((0, 0), (0, 13)))
    pts = jnp.transpose(jnp.concatenate([feats, cost_volume, flow], axis=1),
                        (0, 2, 1)).reshape(_B * _N, 195)
    pts = pts.astype(_BF16).astype(_F32)
    ptsp = jnp.pad(pts, ((0, 0), (0, 13)))

    xyzg, fg1 = _sc_gather(idx_flat, (xyzp, ptsp))

    wn1 = _wn_weights(pc1_wn_w0, pc1_wn_b0, pc1_wn_w1, pc1_wn_b1,
                      pc1_wn_w2, pc1_wn_b2)
    lf1, lx1, lb1 = _lin_weights(pc1_lin_w, pc1_lin_b, 198, 208)
    wn2 = _wn_weights(pc2_wn_w0, pc2_wn_b0, pc2_wn_w1, pc2_wn_b1,
                      pc2_wn_w2, pc2_wn_b2)
    lf2, lx2, lb2 = _lin_weights(pc2_lin_w, pc2_lin_b, 131, 128)

    ctr = xyzp.reshape(_B, _N, 16)
    grid = (_B, _N // _TN)
    out1 = pl.pallas_call(
        _pc1_body,
        grid=grid,
        in_specs=_pc_specs(208),
        out_specs=pl.BlockSpec((1, _TN, 128), lambda b, i: (b, i, 0)),
        out_shape=jax.ShapeDtypeStruct((_B, _N, 128), _F32),
    )(fg1.reshape(_B, _N, _K, 208), xyzg.reshape(_B, _N, _K, 16), ctr,
      lf1, lx1, *wn1, lb1)

    (fg2,) = _sc_gather(idx_flat, (out1.reshape(_B * _N, 128),))

    m0 = mlp0_w.astype(_BF16)
    m1 = mlp1_w.astype(_BF16)
    fcw = fc_w.astype(_BF16)
    xo, flo = pl.pallas_call(
        _pc2_body,
        grid=grid,
        in_specs=_pc_specs(128) + [
            _full((128, 128)), _full((128, 1)),
            _full((64, 128)), _full((64, 1)),
            _full((3, 64)), _full((3, 1)),
        ],
        out_specs=[
            pl.BlockSpec((1, 64, _TN), lambda b, i: (b, 0, i)),
            pl.BlockSpec((1, 3, _TN), lambda b, i: (b, 0, i)),
        ],
        out_shape=[
            jax.ShapeDtypeStruct((_B, 64, _N), _F32),
            jax.ShapeDtypeStruct((_B, 3, _N), _F32),
        ],
    )(fg2.reshape(_B, _N, _K, 128), xyzg.reshape(_B, _N, _K, 16), ctr,
      lf2, lx2, *wn2, lb2,
      m0, mlp0_b.reshape(128, 1), m1, mlp1_b.reshape(64, 1),
      fcw, fc_b.reshape(3, 1))

    return (xo, flo)
